# initial kernel scaffold (unmeasured)
import jax
import jax.numpy as jnp
from jax import lax
from jax.experimental import pallas as pl
from jax.experimental.pallas import tpu as pltpu

N_DEV = 32
B = 64


def _a2a_pallas(send_buf, counts_row):
    _, _, n = send_buf.shape

    def body(send_ref, cnt_ref, recv_ref, rcnt_ref,
             send_sems, recv_sems, csend_sems, crecv_sems):
        i = lax.axis_index("i")

        recv_ref[pl.ds(i, 1)] = send_ref[pl.ds(i, 1)]
        rcnt_ref[pl.ds(i, 1), :] = cnt_ref[:, :]

        send_descs = []
        for k in range(1, N_DEV):
            t = (i + k) % N_DEV
            d = pltpu.make_async_remote_copy(
                src_ref=send_ref.at[t],
                dst_ref=recv_ref.at[i],
                send_sem=send_sems.at[t],
                recv_sem=recv_sems.at[i],
                device_id=(t,),
                device_id_type=pl.DeviceIdType.MESH,
            )
            d.start()
            send_descs.append(d)
            c = pltpu.make_async_remote_copy(
                src_ref=cnt_ref,
                dst_ref=rcnt_ref.at[pl.ds(i, 1), :],
                send_sem=csend_sems.at[t],
                recv_sem=crecv_sems.at[i],
                device_id=(t,),
                device_id_type=pl.DeviceIdType.MESH,
            )
            c.start()
            send_descs.append(c)

        for k in range(1, N_DEV):
            s = (i + k) % N_DEV
            rd = pltpu.make_async_remote_copy(
                src_ref=send_ref.at[s],
                dst_ref=recv_ref.at[s],
                send_sem=send_sems.at[s],
                recv_sem=recv_sems.at[s],
                device_id=(s,),
                device_id_type=pl.DeviceIdType.MESH,
            )
            rd.wait_recv()
            rc = pltpu.make_async_remote_copy(
                src_ref=cnt_ref,
                dst_ref=rcnt_ref.at[pl.ds(s, 1), :],
                send_sem=csend_sems.at[s],
                recv_sem=crecv_sems.at[s],
                device_id=(s,),
                device_id_type=pl.DeviceIdType.MESH,
            )
            rc.wait_recv()

        for d in send_descs:
            d.wait_send()

    return pl.pallas_call(
        body,
        out_shape=[
            jax.ShapeDtypeStruct((N_DEV, B, n), jnp.bfloat16),
            jax.ShapeDtypeStruct((N_DEV, N_DEV), jnp.int32),
        ],
        in_specs=[
            pl.BlockSpec(memory_space=pltpu.VMEM),
            pl.BlockSpec(memory_space=pltpu.VMEM),
        ],
        out_specs=[
            pl.BlockSpec(memory_space=pltpu.VMEM),
            pl.BlockSpec(memory_space=pltpu.VMEM),
        ],
        scratch_shapes=[
            pltpu.SemaphoreType.DMA((N_DEV,)),
            pltpu.SemaphoreType.DMA((N_DEV,)),
            pltpu.SemaphoreType.DMA((N_DEV,)),
            pltpu.SemaphoreType.DMA((N_DEV,)),
        ],
        compiler_params=pltpu.CompilerParams(collective_id=0),
    )(send_buf, counts_row)


def kernel(x, dest):
    m, n = x.shape
    i = lax.axis_index("i")

    counts = jnp.bincount(dest, length=N_DEV).astype(jnp.int32)
    excl = (jnp.cumsum(counts) - counts).astype(jnp.int32)
    order = jnp.argsort(dest, stable=True)
    xs = x[order].astype(jnp.bfloat16)
    xs_pad = jnp.concatenate([xs, jnp.zeros((B, n), jnp.bfloat16)], axis=0)
    row_idx = excl[:, None] + jnp.arange(B, dtype=jnp.int32)[None, :]
    send_buf = xs_pad[row_idx]

    recv_buf, rcnt = _a2a_pallas(send_buf, counts.reshape(1, N_DEV))

    rc = lax.dynamic_index_in_dim(rcnt, i, axis=1, keepdims=False)
    incl = jnp.cumsum(rc)
    rexcl = (incl - rc).astype(jnp.int32)
    r = jnp.arange(m, dtype=jnp.int32)
    s = jnp.searchsorted(incl, r, side="right").astype(jnp.int32)
    k = r - rexcl[s]
    out = recv_buf.reshape(N_DEV * B, n)[s * B + k]
    return out.astype(jnp.float32)


# baseline (device time: 594489 ns/iter reference)
import jax
import jax.numpy as jnp
from jax import lax
from jax.experimental import pallas as pl
from jax.experimental.pallas import tpu as pltpu

N_DEV = 32
B = 64


def _a2a_pallas(send_buf, counts_row):
    _, _, n = send_buf.shape

    def body(send_ref, cnt_ref, recv_ref, rcnt_ref,
             send_sems, recv_sems, csend_sems, crecv_sems):
        i = lax.axis_index("i")

        recv_ref[pl.ds(i, 1)] = send_ref[pl.ds(i, 1)]
        rcnt_ref[pl.ds(i, 1), :] = cnt_ref[:, :]

        send_descs = []
        for k in range(1, N_DEV):
            t = (i + k) % N_DEV
            d = pltpu.make_async_remote_copy(
                src_ref=send_ref.at[t],
                dst_ref=recv_ref.at[i],
                send_sem=send_sems.at[t],
                recv_sem=recv_sems.at[i],
                device_id=(t,),
                device_id_type=pl.DeviceIdType.MESH,
            )
            d.start()
            send_descs.append(d)
            c = pltpu.make_async_remote_copy(
                src_ref=cnt_ref,
                dst_ref=rcnt_ref.at[pl.ds(i, 1), :],
                send_sem=csend_sems.at[t],
                recv_sem=crecv_sems.at[i],
                device_id=(t,),
                device_id_type=pl.DeviceIdType.MESH,
            )
            c.start()
            send_descs.append(c)

        for k in range(1, N_DEV):
            s = (i + k) % N_DEV
            rd = pltpu.make_async_remote_copy(
                src_ref=send_ref.at[s],
                dst_ref=recv_ref.at[s],
                send_sem=send_sems.at[s],
                recv_sem=recv_sems.at[s],
                device_id=(s,),
                device_id_type=pl.DeviceIdType.MESH,
            )
            rd.wait_recv()
            rc = pltpu.make_async_remote_copy(
                src_ref=cnt_ref,
                dst_ref=rcnt_ref.at[pl.ds(s, 1), :],
                send_sem=csend_sems.at[s],
                recv_sem=crecv_sems.at[s],
                device_id=(s,),
                device_id_type=pl.DeviceIdType.MESH,
            )
            rc.wait_recv()

        for d in send_descs:
            d.wait_send()

    return pl.pallas_call(
        body,
        out_shape=[
            jax.ShapeDtypeStruct((N_DEV, B, n), jnp.bfloat16),
            jax.ShapeDtypeStruct((N_DEV, N_DEV), jnp.int32),
        ],
        in_specs=[
            pl.BlockSpec(memory_space=pltpu.VMEM),
            pl.BlockSpec(memory_space=pltpu.VMEM),
        ],
        out_specs=[
            pl.BlockSpec(memory_space=pltpu.VMEM),
            pl.BlockSpec(memory_space=pltpu.VMEM),
        ],
        scratch_shapes=[
            pltpu.SemaphoreType.DMA((N_DEV,)),
            pltpu.SemaphoreType.DMA((N_DEV,)),
            pltpu.SemaphoreType.DMA((N_DEV,)),
            pltpu.SemaphoreType.DMA((N_DEV,)),
        ],
    )(send_buf, counts_row)


def kernel(x, dest):
    m, n = x.shape
    i = lax.axis_index("i")

    counts = jnp.bincount(dest, length=N_DEV).astype(jnp.int32)
    excl = (jnp.cumsum(counts) - counts).astype(jnp.int32)
    order = jnp.argsort(dest, stable=True)
    xs = x[order].astype(jnp.bfloat16)
    xs_pad = jnp.concatenate([xs, jnp.zeros((B, n), jnp.bfloat16)], axis=0)
    row_idx = excl[:, None] + jnp.arange(B, dtype=jnp.int32)[None, :]
    send_buf = xs_pad[row_idx]

    recv_buf, rcnt = _a2a_pallas(send_buf, counts.reshape(1, N_DEV))

    rc = lax.dynamic_index_in_dim(rcnt, i, axis=1, keepdims=False)
    incl = jnp.cumsum(rc)
    rexcl = (incl - rc).astype(jnp.int32)
    r = jnp.arange(m, dtype=jnp.int32)
    s = jnp.searchsorted(incl, r, side="right").astype(jnp.int32)
    k = r - rexcl[s]
    out = recv_buf.reshape(N_DEV * B, n)[s * B + k]
    return out.astype(jnp.float32)


# device time: 47838 ns/iter; 12.4271x vs baseline; 12.4271x over previous
import jax
import jax.numpy as jnp
from jax import lax
from jax.experimental import pallas as pl
from jax.experimental.pallas import tpu as pltpu

N_DEV = 32
B = 64


def _a2a_pallas(send_buf, counts_row):
    _, _, n = send_buf.shape

    def body(send_ref, cnt_ref, recv_ref, rcnt_ref,
             send_sems, recv_sems, csend_sems, crecv_sems):
        i = lax.axis_index("i")

        recv_ref[pl.ds(i, 1)] = send_ref[pl.ds(i, 1)]
        rcnt_ref[pl.ds(i, 1), :] = cnt_ref[:, :]

        send_descs = []
        for k in range(1, N_DEV):
            t = (i + k) % N_DEV
            d = pltpu.make_async_remote_copy(
                src_ref=send_ref.at[t],
                dst_ref=recv_ref.at[i],
                send_sem=send_sems.at[t],
                recv_sem=recv_sems.at[i],
                device_id=(t,),
                device_id_type=pl.DeviceIdType.MESH,
            )
            d.start()
            send_descs.append(d)
            c = pltpu.make_async_remote_copy(
                src_ref=cnt_ref,
                dst_ref=rcnt_ref.at[pl.ds(i, 1), :],
                send_sem=csend_sems.at[t],
                recv_sem=crecv_sems.at[i],
                device_id=(t,),
                device_id_type=pl.DeviceIdType.MESH,
            )
            c.start()
            send_descs.append(c)

        for k in range(1, N_DEV):
            s = (i + k) % N_DEV
            rd = pltpu.make_async_remote_copy(
                src_ref=send_ref.at[s],
                dst_ref=recv_ref.at[s],
                send_sem=send_sems.at[s],
                recv_sem=recv_sems.at[s],
                device_id=(s,),
                device_id_type=pl.DeviceIdType.MESH,
            )
            rd.wait_recv()
            rc = pltpu.make_async_remote_copy(
                src_ref=cnt_ref,
                dst_ref=rcnt_ref.at[pl.ds(s, 1), :],
                send_sem=csend_sems.at[s],
                recv_sem=crecv_sems.at[s],
                device_id=(s,),
                device_id_type=pl.DeviceIdType.MESH,
            )
            rc.wait_recv()

        for d in send_descs:
            d.wait_send()

    return pl.pallas_call(
        body,
        out_shape=[
            jax.ShapeDtypeStruct((N_DEV, B, n), jnp.bfloat16),
            jax.ShapeDtypeStruct((N_DEV, N_DEV), jnp.int32),
        ],
        in_specs=[
            pl.BlockSpec(memory_space=pltpu.VMEM),
            pl.BlockSpec(memory_space=pltpu.VMEM),
        ],
        out_specs=[
            pl.BlockSpec(memory_space=pltpu.VMEM),
            pl.BlockSpec(memory_space=pltpu.VMEM),
        ],
        scratch_shapes=[
            pltpu.SemaphoreType.DMA((N_DEV,)),
            pltpu.SemaphoreType.DMA((N_DEV,)),
            pltpu.SemaphoreType.DMA((N_DEV,)),
            pltpu.SemaphoreType.DMA((N_DEV,)),
        ],
    )(send_buf, counts_row)


def kernel(x, dest):
    m, n = x.shape
    i = lax.axis_index("i")

    j = jnp.arange(m, dtype=jnp.int32)
    same = dest[:, None] == dest[None, :]
    earlier = j[None, :] < j[:, None]
    rank = jnp.sum(same & earlier, axis=1).astype(jnp.int32)
    slot = dest.astype(jnp.int32) * B + rank
    M = (jnp.arange(N_DEV * B, dtype=jnp.int32)[:, None] == slot[None, :])
    send_flat = jnp.dot(
        M.astype(jnp.bfloat16), x.astype(jnp.bfloat16),
        preferred_element_type=jnp.bfloat16,
    )
    send_buf = send_flat.reshape(N_DEV, B, n)
    counts = jnp.sum(
        dest[None, :] == jnp.arange(N_DEV, dtype=jnp.int32)[:, None], axis=1
    ).astype(jnp.int32)

    recv_buf, rcnt = _a2a_pallas(send_buf, counts.reshape(1, N_DEV))

    rc = lax.dynamic_index_in_dim(rcnt, i, axis=1, keepdims=False)
    incl = jnp.cumsum(rc)
    rexcl = (incl - rc).astype(jnp.int32)
    r = jnp.arange(m, dtype=jnp.int32)
    srcidx = jnp.sum(r[:, None] >= incl[None, :], axis=1).astype(jnp.int32)
    src_onehot = srcidx[:, None] == jnp.arange(N_DEV, dtype=jnp.int32)[None, :]
    rexcl_at = jnp.sum(src_onehot * rexcl[None, :], axis=1)
    flatpos = srcidx * B + r - rexcl_at
    Q = (flatpos[:, None] == jnp.arange(N_DEV * B, dtype=jnp.int32)[None, :])
    out = jnp.dot(
        Q.astype(jnp.bfloat16), recv_buf.reshape(N_DEV * B, n),
        preferred_element_type=jnp.float32,
    )
    return out
